# baseline (speedup 1.0000x reference)
import jax
import jax.numpy as jnp
from jax.experimental import pallas as pl
from jax.experimental.pallas import tpu as pltpu

IN_DIMS = 1024
SEQ_LEN = 2048
TOP_K = 64
BLK = 256
N_BISECT = 10
NBLK = SEQ_LEN // BLK


def _fused_body(xf_ref, w_ref, b_ref, out_ref, xn_ref, sc_ref):
    j = pl.program_id(1)

    @pl.when(j == 0)
    def _init():
        xf32 = xf_ref[0].astype(jnp.float32)
        rn = jax.lax.rsqrt(jnp.maximum(
            jnp.sum(xf32 * xf32, axis=1, keepdims=True), 1e-24))
        xn_ref[...] = (xf32 * rn).astype(jnp.bfloat16)

    par = jax.lax.rem(j, 2)

    ip = jnp.minimum(j, NBLK - 1)
    xnb = xn_ref[pl.ds(ip * BLK, BLK), :]
    sc_ref[par] = jax.lax.dot_general(
        xn_ref[...], xnb, (((1,), (1,)), ((), ())),
        preferred_element_type=jnp.float32)

    scores = sc_ref[1 - par]

    lo = jnp.full((1, BLK), -1.01, jnp.float32)
    hi = jnp.full((1, BLK), 1.01, jnp.float32)
    for _ in range(N_BISECT):
        mid = 0.5 * (lo + hi)
        cnt = jnp.sum((scores >= mid).astype(jnp.float32), axis=0,
                      keepdims=True)
        ge = cnt >= TOP_K
        lo, hi = jnp.where(ge, mid, lo), jnp.where(ge, hi, mid)

    e = jnp.exp(scores)
    a = jnp.where(scores >= lo, e, 1.0)
    denom = jnp.sum(a, axis=0, keepdims=True)
    attn = (a * (1.0 / denom)).astype(jnp.bfloat16)

    g = jax.lax.dot_general(
        attn, xf_ref[0], (((0,), (0,)), ((), ())),
        preferred_element_type=jnp.float32)
    out = jax.lax.dot_general(
        g.astype(jnp.bfloat16), w_ref[...], (((1,), (1,)), ((), ())),
        preferred_element_type=jnp.float32)
    out_ref[0] = out + b_ref[...]


def kernel(x, W, b):
    B, S, D = x.shape
    b2 = b.reshape(1, D)
    xb16 = x.astype(jnp.bfloat16)
    Wb16 = W.astype(jnp.bfloat16)
    out = pl.pallas_call(
        _fused_body,
        grid=(B, NBLK + 1),
        in_specs=[
            pl.BlockSpec((1, S, D), lambda bi, j: (bi, 0, 0)),
            pl.BlockSpec((D, D), lambda bi, j: (0, 0)),
            pl.BlockSpec((1, D), lambda bi, j: (0, 0)),
        ],
        out_specs=pl.BlockSpec(
            (1, BLK, D), lambda bi, j: (bi, jnp.maximum(j - 1, 0), 0)),
        out_shape=jax.ShapeDtypeStruct((B, S, D), jnp.float32),
        scratch_shapes=[
            pltpu.VMEM((S, D), jnp.bfloat16),
            pltpu.VMEM((2, S, BLK), jnp.float32),
        ],
    )(xb16, Wb16, b2)
    return out


# speedup vs baseline: 1.6761x; 1.6761x over previous
import jax
import jax.numpy as jnp
from jax.experimental import pallas as pl
from jax.experimental.pallas import tpu as pltpu

IN_DIMS = 1024
SEQ_LEN = 2048
TOP_K = 64
BLK = 256
N_BISECT = 10


def _process(scores, xf, w, b):
    lo = jnp.full((1, BLK), -1.01, jnp.float32)
    hi = jnp.full((1, BLK), 1.01, jnp.float32)
    for _ in range(N_BISECT):
        mid = 0.5 * (lo + hi)
        cnt = jnp.sum((scores >= mid).astype(jnp.float32), axis=0,
                      keepdims=True)
        ge = cnt >= TOP_K
        lo, hi = jnp.where(ge, mid, lo), jnp.where(ge, hi, mid)

    e = jnp.exp(scores)
    a = jnp.where(scores >= lo, e, 1.0)
    denom = jnp.sum(a, axis=0, keepdims=True)
    attn = (a * (1.0 / denom)).astype(jnp.bfloat16)

    g = jax.lax.dot_general(
        attn, xf, (((0,), (0,)), ((), ())),
        preferred_element_type=jnp.float32)
    out = jax.lax.dot_general(
        g.astype(jnp.bfloat16), w, (((1,), (1,)), ((), ())),
        preferred_element_type=jnp.float32)
    return out + b


def _fused_body(xf_ref, w_ref, b_ref, out_ref, xn_ref):
    j = pl.program_id(1)

    @pl.when(j == 0)
    def _init():
        xf32 = xf_ref[0].astype(jnp.float32)
        rn = jax.lax.rsqrt(jnp.maximum(
            jnp.sum(xf32 * xf32, axis=1, keepdims=True), 1e-24))
        xn_ref[...] = (xf32 * rn).astype(jnp.bfloat16)

    xn = xn_ref[...]
    base = j * (2 * BLK)
    xnb_a = xn_ref[pl.ds(base, BLK), :]
    xnb_b = xn_ref[pl.ds(base + BLK, BLK), :]
    scores_a = jax.lax.dot_general(
        xn, xnb_a, (((1,), (1,)), ((), ())),
        preferred_element_type=jnp.float32)
    scores_b = jax.lax.dot_general(
        xn, xnb_b, (((1,), (1,)), ((), ())),
        preferred_element_type=jnp.float32)

    xf = xf_ref[0]
    w = w_ref[...]
    b = b_ref[...]
    out_ref[0, :BLK, :] = _process(scores_a, xf, w, b)
    out_ref[0, BLK:, :] = _process(scores_b, xf, w, b)


def kernel(x, W, b):
    B, S, D = x.shape
    b2 = b.reshape(1, D)
    xb16 = x.astype(jnp.bfloat16)
    Wb16 = W.astype(jnp.bfloat16)
    out = pl.pallas_call(
        _fused_body,
        grid=(B, S // (2 * BLK)),
        in_specs=[
            pl.BlockSpec((1, S, D), lambda bi, j: (bi, 0, 0)),
            pl.BlockSpec((D, D), lambda bi, j: (0, 0)),
            pl.BlockSpec((1, D), lambda bi, j: (0, 0)),
        ],
        out_specs=pl.BlockSpec((1, 2 * BLK, D), lambda bi, j: (bi, j, 0)),
        out_shape=jax.ShapeDtypeStruct((B, S, D), jnp.float32),
        scratch_shapes=[
            pltpu.VMEM((S, D), jnp.bfloat16),
        ],
    )(xb16, Wb16, b2)
    return out


# speedup vs baseline: 1.8461x; 1.1014x over previous
import jax
import jax.numpy as jnp
from jax.experimental import pallas as pl
from jax.experimental.pallas import tpu as pltpu

IN_DIMS = 1024
SEQ_LEN = 2048
TOP_K = 64
BLK = 256
N_BISECT = 10
SUB = 4


def _process(scores, xf, w, b):
    lo = jnp.full((1, BLK), -1.01, jnp.float32)
    hi = jnp.full((1, BLK), 1.01, jnp.float32)
    for _ in range(N_BISECT):
        mid = 0.5 * (lo + hi)
        cnt = jnp.sum((scores >= mid).astype(jnp.float32), axis=0,
                      keepdims=True)
        ge = cnt >= TOP_K
        lo, hi = jnp.where(ge, mid, lo), jnp.where(ge, hi, mid)

    e = jnp.exp(scores)
    a = jnp.where(scores >= lo, e, 1.0)
    denom = jnp.sum(a, axis=0, keepdims=True)
    attn = (a * (1.0 / denom)).astype(jnp.bfloat16)

    g = jax.lax.dot_general(
        attn, xf, (((0,), (0,)), ((), ())),
        preferred_element_type=jnp.float32)
    out = jax.lax.dot_general(
        g.astype(jnp.bfloat16), w, (((1,), (1,)), ((), ())),
        preferred_element_type=jnp.float32)
    return out + b


def _fused_body(xf_ref, w_ref, b_ref, out_ref, xn_ref):
    j = pl.program_id(1)

    @pl.when(j == 0)
    def _init():
        xf32 = xf_ref[0].astype(jnp.float32)
        rn = jax.lax.rsqrt(jnp.maximum(
            jnp.sum(xf32 * xf32, axis=1, keepdims=True), 1e-24))
        xn_ref[...] = (xf32 * rn).astype(jnp.bfloat16)

    xn = xn_ref[...]
    base = j * (SUB * BLK)
    scores = [
        jax.lax.dot_general(
            xn, xn_ref[pl.ds(base + k * BLK, BLK), :],
            (((1,), (1,)), ((), ())),
            preferred_element_type=jnp.float32)
        for k in range(SUB)
    ]

    xf = xf_ref[0]
    w = w_ref[...]
    b = b_ref[...]
    for k in range(SUB):
        out_ref[0, k * BLK:(k + 1) * BLK, :] = _process(scores[k], xf, w, b)


def kernel(x, W, b):
    B, S, D = x.shape
    b2 = b.reshape(1, D)
    xb16 = x.astype(jnp.bfloat16)
    Wb16 = W.astype(jnp.bfloat16)
    out = pl.pallas_call(
        _fused_body,
        grid=(B, S // (SUB * BLK)),
        in_specs=[
            pl.BlockSpec((1, S, D), lambda bi, j: (bi, 0, 0)),
            pl.BlockSpec((D, D), lambda bi, j: (0, 0)),
            pl.BlockSpec((1, D), lambda bi, j: (0, 0)),
        ],
        out_specs=pl.BlockSpec((1, SUB * BLK, D), lambda bi, j: (bi, j, 0)),
        out_shape=jax.ShapeDtypeStruct((B, S, D), jnp.float32),
        scratch_shapes=[
            pltpu.VMEM((S, D), jnp.bfloat16),
        ],
    )(xb16, Wb16, b2)
    return out
